# Initial kernel scaffold; baseline (speedup 1.0000x reference)
import jax
import jax.numpy as jnp
from jax.experimental import pallas as pl


def kernel(x, edge_index, W, b, gamma, beta):
    raise NotImplementedError("write your pallas kernel here")


# speedup vs baseline: 10.3614x; 10.3614x over previous
import functools

import jax
import jax.numpy as jnp
from jax import lax
from jax.experimental import pallas as pl
from jax.experimental.pallas import tpu as pltpu
from jax.experimental.pallas import tpu_sc as plsc

NC = 2
NS = 16
NW = NC * NS
K = 80


def _scatter_rows_kernel(N_pad, D, E):
    e_w = E // NW
    n_chunk = e_w // K
    rows_per_tile = N_pad // NS
    nz = rows_per_tile // K

    mesh = plsc.VectorSubcoreMesh(core_axis_name="c", subcore_axis_name="s")

    @functools.partial(
        pl.kernel,
        out_type=jax.ShapeDtypeStruct((NC, N_pad, D), jnp.float32),
        mesh=mesh,
        scratch_types=[
            pltpu.VMEM_SHARED((N_pad, D), jnp.float32),
            pltpu.VMEM((K,), jnp.int32),
            pltpu.VMEM((K,), jnp.int32),
            pltpu.VMEM((K, D), jnp.float32),
        ],
    )
    def scat(u_hbm, src_hbm, dst_hbm, zeros_hbm, out_hbm, acc, srcv, dstv, rows):
        c = lax.axis_index("c")
        s = lax.axis_index("s")
        row0 = s * rows_per_tile

        pltpu.sync_copy(zeros_hbm, rows)
        for j in range(nz):
            pltpu.sync_copy(rows, acc.at[pl.ds(row0 + j * K, K)])
        plsc.subcore_barrier()

        base = (c * NS + s) * e_w

        def body(i, _):
            off = base + i * K
            pltpu.sync_copy(src_hbm.at[pl.ds(off, K)], srcv)
            pltpu.sync_copy(dst_hbm.at[pl.ds(off, K)], dstv)
            pltpu.sync_copy(u_hbm.at[srcv], rows)
            pltpu.sync_copy(rows, acc.at[dstv], add=True)
            return _

        lax.fori_loop(0, n_chunk, body, None)
        plsc.subcore_barrier()

        for j in range(nz):
            pltpu.sync_copy(acc.at[pl.ds(row0 + j * K, K)], rows)
            pltpu.sync_copy(rows, out_hbm.at[c, pl.ds(row0 + j * K, K)])

    return scat


def _degree_kernel(N_pad, E):
    e_w = E // NW
    n_chunk = e_w // K
    words_per_tile = N_pad // NS

    mesh = plsc.VectorSubcoreMesh(core_axis_name="c", subcore_axis_name="s")

    @functools.partial(
        pl.kernel,
        out_type=jax.ShapeDtypeStruct((NC, N_pad), jnp.float32),
        mesh=mesh,
        scratch_types=[
            pltpu.VMEM_SHARED((N_pad,), jnp.float32),
            pltpu.VMEM((K,), jnp.int32),
            pltpu.VMEM((K,), jnp.float32),
            pltpu.VMEM((words_per_tile,), jnp.float32),
        ],
    )
    def degk(dst_hbm, ones_hbm, zeros_hbm, out_hbm, acc, dstv, onesv, zbuf):
        c = lax.axis_index("c")
        s = lax.axis_index("s")
        w0 = s * words_per_tile

        pltpu.sync_copy(zeros_hbm, zbuf)
        pltpu.sync_copy(zbuf, acc.at[pl.ds(w0, words_per_tile)])
        pltpu.sync_copy(ones_hbm, onesv)
        plsc.subcore_barrier()

        base = (c * NS + s) * e_w

        def body(i, _):
            off = base + i * K
            pltpu.sync_copy(dst_hbm.at[pl.ds(off, K)], dstv)
            pltpu.sync_copy(onesv, acc.at[dstv], add=True)
            return _

        lax.fori_loop(0, n_chunk, body, None)
        plsc.subcore_barrier()

        pltpu.sync_copy(acc.at[pl.ds(w0, words_per_tile)], zbuf)
        pltpu.sync_copy(zbuf, out_hbm.at[c, pl.ds(w0, words_per_tile)])

    return degk


def _pre_tc(N, D, N_pad):
    def body(cnt_ref, x_ref, w_ref, dinv_ref, u_ref):
        deg = cnt_ref[0, :N] + cnt_ref[1, :N] + 1.0
        dinv = lax.rsqrt(deg)
        dinv_ref[...] = dinv
        xw = jnp.dot(x_ref[...], w_ref[...], preferred_element_type=jnp.float32)
        u_ref[...] = xw * dinv[:, None]

    return pl.pallas_call(
        body,
        out_shape=(
            jax.ShapeDtypeStruct((N,), jnp.float32),
            jax.ShapeDtypeStruct((N, D), jnp.float32),
        ),
    )


def _layer_tc(N, D, N_pad, last):
    eps = 1e-5

    def body(S_ref, u_ref, dinv_ref, b_ref, g_ref, be_ref, w_ref, out_ref):
        u = u_ref[...]
        dinv = dinv_ref[...]
        S = S_ref[0, :N, :] + S_ref[1, :N, :] + u
        agg = S * dinv[:, None] + b_ref[...]
        mean = jnp.mean(agg, axis=0)
        var = jnp.mean((agg - mean[None, :]) ** 2, axis=0)
        h = (agg - mean[None, :]) * lax.rsqrt(var + eps)
        h = h * g_ref[...] + be_ref[...]
        h = jnp.maximum(h, 0.0)
        if last:
            out_ref[...] = h
        else:
            hw = jnp.dot(h, w_ref[...], preferred_element_type=jnp.float32)
            out_ref[...] = hw * dinv[:, None]

    return pl.pallas_call(
        body,
        out_shape=jax.ShapeDtypeStruct((N, D), jnp.float32),
    )


def kernel(x, edge_index, W, b, gamma, beta):
    N, D = x.shape
    E = edge_index.shape[1]
    L = W.shape[0]
    assert E % (NW * K) == 0
    N_pad = ((N + NS * K - 1) // (NS * K)) * (NS * K)

    src = edge_index[0]
    dst = edge_index[1]
    zeros_rows = jnp.zeros((K, D), jnp.float32)
    zeros_deg = jnp.zeros((N_pad // NS,), jnp.float32)
    ones_k = jnp.ones((K,), jnp.float32)

    cnt = _degree_kernel(N_pad, E)(dst, ones_k, zeros_deg)
    dinv, u = _pre_tc(N, D, N_pad)(cnt, x, W[0])

    scat = _scatter_rows_kernel(N_pad, D, E)
    for i in range(L):
        S = scat(u, src, dst, zeros_rows)
        layer = _layer_tc(N, D, N_pad, last=(i == L - 1))
        w_next = W[i + 1] if i < L - 1 else W[0]
        u = layer(S, u, dinv, b[i].reshape(1, D), gamma[i].reshape(1, D),
                  beta[i].reshape(1, D), w_next)
    return u


# speedup vs baseline: 21.2456x; 2.0504x over previous
import functools

import jax
import jax.numpy as jnp
from jax import lax
from jax.experimental import pallas as pl
from jax.experimental.pallas import tpu as pltpu
from jax.experimental.pallas import tpu_sc as plsc

NC = 2
NS = 16
NW = NC * NS
K = 80
K2 = 128
KZ = 80


def _scatter_rows_kernel(N_pad, D, E_pad):
    e_w = E_pad // NW
    n_chunk = e_w // K2
    rows_per_tile = N_pad // NS
    nz = rows_per_tile // KZ
    assert rows_per_tile % KZ == 0 and e_w % K2 == 0 and KZ <= K2
    assert n_chunk % 2 == 0 and n_chunk >= 4

    mesh = plsc.VectorSubcoreMesh(core_axis_name="c", subcore_axis_name="s")

    @functools.partial(
        pl.kernel,
        out_type=jax.ShapeDtypeStruct((NC, N_pad, D), jnp.float32),
        mesh=mesh,
        scratch_types=[
            pltpu.VMEM_SHARED((N_pad, D), jnp.float32),
            pltpu.VMEM((2, K2), jnp.int32),
            pltpu.VMEM((2, K2), jnp.int32),
            pltpu.VMEM((2, K2, D), jnp.float32),
            pltpu.SemaphoreType.DMA,
            pltpu.SemaphoreType.DMA,
            pltpu.SemaphoreType.DMA,
            pltpu.SemaphoreType.DMA,
        ],
    )
    def scat(u_hbm, src_hbm, dst_hbm, zeros_hbm, out_hbm,
             acc, sidx, didx, ring, gs0, gs1, is0, is1):
        gsem = (gs0, gs1)
        isem = (is0, is1)
        c = lax.axis_index("c")
        s = lax.axis_index("s")
        wid = c * NS + s
        row0 = s * rows_per_tile

        zchunk = ring.at[0, pl.ds(0, KZ)]
        pltpu.sync_copy(zeros_hbm, zchunk)
        for j in range(nz):
            pltpu.sync_copy(zchunk, acc.at[pl.ds(row0 + j * KZ, KZ)])
        plsc.subcore_barrier()

        def fire_idx(i, b):
            pltpu.async_copy(src_hbm.at[wid, i], sidx.at[b], isem[b])
            pltpu.async_copy(dst_hbm.at[wid, i], didx.at[b], isem[b])

        def wait_idx(i, b):
            pltpu.make_async_copy(src_hbm.at[wid, i], sidx.at[b],
                                  isem[b]).wait()
            pltpu.make_async_copy(dst_hbm.at[wid, i], didx.at[b],
                                  isem[b]).wait()

        def fire_g(b):
            pltpu.async_copy(u_hbm.at[sidx.at[b]], ring.at[b], gsem[b])

        def wait_g(b):
            pltpu.make_async_copy(u_hbm.at[sidx.at[b]], ring.at[b],
                                  gsem[b]).wait()

        def scat_add(b):
            pltpu.sync_copy(ring.at[b], acc.at[didx.at[b]], add=True)

        fire_idx(0, 0)
        fire_idx(1, 1)
        wait_idx(0, 0)
        fire_g(0)

        def slot(i, b):
            wait_g(b)
            wait_idx(i + 1, 1 - b)
            fire_g(1 - b)
            scat_add(b)
            fire_idx(i + 2, b)

        def outer(g, _):
            slot(2 * g, 0)
            slot(2 * g + 1, 1)
            return _

        lax.fori_loop(0, (n_chunk - 2) // 2, outer, None)
        b = 0
        wait_g(b)
        wait_idx(n_chunk - 1, 1 - b)
        fire_g(1 - b)
        scat_add(b)
        wait_g(1 - b)
        scat_add(1 - b)

        plsc.subcore_barrier()

        for j in range(nz):
            pltpu.sync_copy(acc.at[pl.ds(row0 + j * KZ, KZ)], zchunk)
            pltpu.sync_copy(zchunk, out_hbm.at[c, pl.ds(row0 + j * KZ, KZ)])

    return scat


def _degree_kernel(N_pad, E):
    e_w = E // NW
    n_chunk = e_w // K
    words_per_tile = N_pad // NS

    mesh = plsc.VectorSubcoreMesh(core_axis_name="c", subcore_axis_name="s")

    @functools.partial(
        pl.kernel,
        out_type=jax.ShapeDtypeStruct((NC, N_pad), jnp.float32),
        mesh=mesh,
        scratch_types=[
            pltpu.VMEM_SHARED((N_pad,), jnp.float32),
            pltpu.VMEM((K,), jnp.int32),
            pltpu.VMEM((K,), jnp.float32),
            pltpu.VMEM((words_per_tile,), jnp.float32),
        ],
    )
    def degk(dst_hbm, ones_hbm, zeros_hbm, out_hbm, acc, dstv, onesv, zbuf):
        c = lax.axis_index("c")
        s = lax.axis_index("s")
        w0 = s * words_per_tile

        pltpu.sync_copy(zeros_hbm, zbuf)
        pltpu.sync_copy(zbuf, acc.at[pl.ds(w0, words_per_tile)])
        pltpu.sync_copy(ones_hbm, onesv)
        plsc.subcore_barrier()

        base = (c * NS + s) * e_w

        def body(i, _):
            off = base + i * K
            pltpu.sync_copy(dst_hbm.at[pl.ds(off, K)], dstv)
            pltpu.sync_copy(onesv, acc.at[dstv], add=True)
            return _

        lax.fori_loop(0, n_chunk, body, None)
        plsc.subcore_barrier()

        pltpu.sync_copy(acc.at[pl.ds(w0, words_per_tile)], zbuf)
        pltpu.sync_copy(zbuf, out_hbm.at[c, pl.ds(w0, words_per_tile)])

    return degk


def _pre_tc(N, D, N_pad):
    def body(cnt_ref, x_ref, w_ref, dinv_ref, u_ref):
        deg = cnt_ref[0, :N] + cnt_ref[1, :N] + 1.0
        dinv = lax.rsqrt(deg)
        dinv_ref[...] = dinv
        xw = jnp.dot(x_ref[...], w_ref[...], preferred_element_type=jnp.float32)
        u_ref[...] = xw * dinv[:, None]

    return pl.pallas_call(
        body,
        out_shape=(
            jax.ShapeDtypeStruct((N,), jnp.float32),
            jax.ShapeDtypeStruct((N, D), jnp.float32),
        ),
    )


def _layer_tc(N, D, N_pad, last):
    eps = 1e-5

    def body(S_ref, u_ref, dinv_ref, b_ref, g_ref, be_ref, w_ref, out_ref):
        u = u_ref[...]
        dinv = dinv_ref[...]
        S = S_ref[0, :N, :] + S_ref[1, :N, :] + u
        agg = S * dinv[:, None] + b_ref[...]
        mean = jnp.mean(agg, axis=0)
        var = jnp.mean((agg - mean[None, :]) ** 2, axis=0)
        h = (agg - mean[None, :]) * lax.rsqrt(var + eps)
        h = h * g_ref[...] + be_ref[...]
        h = jnp.maximum(h, 0.0)
        if last:
            out_ref[...] = h
        else:
            hw = jnp.dot(h, w_ref[...], preferred_element_type=jnp.float32)
            out_ref[...] = hw * dinv[:, None]

    return pl.pallas_call(
        body,
        out_shape=jax.ShapeDtypeStruct((N, D), jnp.float32),
    )


def kernel(x, edge_index, W, b, gamma, beta):
    N, D = x.shape
    E = edge_index.shape[1]
    L = W.shape[0]
    assert E % (NW * K) == 0
    N_pad = ((N + NS * K - 1) // (NS * K)) * (NS * K)

    src = edge_index[0]
    dst = edge_index[1]
    zeros_rows = jnp.zeros((KZ, D), jnp.float32)
    zeros_deg = jnp.zeros((N_pad // NS,), jnp.float32)
    ones_k = jnp.ones((K,), jnp.float32)

    E_pad = ((E + 2 * NW * K2 - 1) // (2 * NW * K2)) * (2 * NW * K2)
    n_extra = E_pad - E
    pad_iota = jnp.arange(n_extra, dtype=jnp.int32)
    src_pad = jnp.concatenate([src, pad_iota % N]).reshape(NW, -1, K2)
    dst_pad = jnp.concatenate(
        [dst, N + pad_iota % (N_pad - N)]).reshape(NW, -1, K2)

    cnt = _degree_kernel(N_pad, E)(dst, ones_k, zeros_deg)
    dinv, u = _pre_tc(N, D, N_pad)(cnt, x, W[0])

    scat = _scatter_rows_kernel(N_pad, D, E_pad)
    for i in range(L):
        S = scat(u, src_pad, dst_pad, zeros_rows)
        layer = _layer_tc(N, D, N_pad, last=(i == L - 1))
        w_next = W[i + 1] if i < L - 1 else W[0]
        u = layer(S, u, dinv, b[i].reshape(1, D), gamma[i].reshape(1, D),
                  beta[i].reshape(1, D), w_next)
    return u


# speedup vs baseline: 21.4347x; 1.0089x over previous
import functools

import jax
import jax.numpy as jnp
from jax import lax
from jax.experimental import pallas as pl
from jax.experimental.pallas import tpu as pltpu
from jax.experimental.pallas import tpu_sc as plsc

NC = 2
NS = 16
NW = NC * NS
K = 80
K2 = 128
KZ = 80


def _scatter_rows_kernel(N_pad, D, E_pad):
    n_chunk, n_arr = E_pad
    rows_per_tile = N_pad // NS
    nz = rows_per_tile // KZ
    assert rows_per_tile % KZ == 0 and KZ <= K2
    assert n_chunk % 4 == 3 and n_chunk >= 7 and n_arr >= n_chunk

    mesh = plsc.VectorSubcoreMesh(core_axis_name="c", subcore_axis_name="s")

    @functools.partial(
        pl.kernel,
        out_type=jax.ShapeDtypeStruct((NC, N_pad, D), jnp.float32),
        mesh=mesh,
        scratch_types=[
            pltpu.VMEM_SHARED((N_pad, D), jnp.float32),
            pltpu.VMEM((4, K2), jnp.int32),
            pltpu.VMEM((4, K2), jnp.int32),
            pltpu.VMEM((2, K2, D), jnp.float32),
            pltpu.SemaphoreType.DMA,
            pltpu.SemaphoreType.DMA,
            pltpu.SemaphoreType.DMA,
            pltpu.SemaphoreType.DMA,
            pltpu.SemaphoreType.DMA,
            pltpu.SemaphoreType.DMA,
            pltpu.SemaphoreType.DMA,
            pltpu.SemaphoreType.DMA,
        ],
    )
    def scat(u_hbm, src_hbm, dst_hbm, zeros_hbm, out_hbm,
             acc, sidx, didx, ring,
             gs0, gs1, ss0, ss1, is0, is1, is2, is3):
        gsem = (gs0, gs1)
        ssem = (ss0, ss1)
        isem = (is0, is1, is2, is3)
        c = lax.axis_index("c")
        s = lax.axis_index("s")
        wid = c * NS + s
        row0 = s * rows_per_tile

        zchunk = ring.at[0, pl.ds(0, KZ)]
        pltpu.sync_copy(zeros_hbm, zchunk)
        for j in range(nz):
            pltpu.sync_copy(zchunk, acc.at[pl.ds(row0 + j * KZ, KZ)])
        plsc.subcore_barrier()

        def fire_idx(i, ib):
            pltpu.async_copy(src_hbm.at[wid, i], sidx.at[ib], isem[ib])
            pltpu.async_copy(dst_hbm.at[wid, i], didx.at[ib], isem[ib])

        def wait_idx(i, ib):
            pltpu.make_async_copy(src_hbm.at[wid, i], sidx.at[ib],
                                  isem[ib]).wait()
            pltpu.make_async_copy(dst_hbm.at[wid, i], didx.at[ib],
                                  isem[ib]).wait()

        def fire_g(ib, db):
            pltpu.async_copy(u_hbm.at[sidx.at[ib]], ring.at[db], gsem[db])

        def wait_g(ib, db):
            pltpu.make_async_copy(u_hbm.at[sidx.at[ib]], ring.at[db],
                                  gsem[db]).wait()

        def fire_scat(ib, db):
            pltpu.async_copy(ring.at[db], acc.at[didx.at[ib]], ssem[db],
                             add=True)

        def wait_scat(ib, db):
            pltpu.make_async_copy(ring.at[db], acc.at[didx.at[ib]],
                                  ssem[db]).wait()

        def slot(i, ib, has_prev=True, fire_next=True):
            db = ib % 2
            wait_g(ib, db)
            fire_scat(ib, db)
            if has_prev:
                wait_scat((ib - 1) % 4, 1 - db)
            wait_idx(i + 1, (ib + 1) % 4)
            fire_g((ib + 1) % 4, 1 - db)
            if fire_next:
                fire_idx(i + 2, (ib + 2) % 4)

        fire_idx(0, 0)
        fire_idx(1, 1)
        wait_idx(0, 0)
        fire_g(0, 0)
        slot(0, 0, has_prev=False)
        def outer(g, _):
            for bb in range(4):
                i = 4 * g + 1 + bb
                slot(i, (1 + bb) % 4)
            return _

        lax.fori_loop(0, (n_chunk - 3) // 4, outer, None)
        i = n_chunk - 2
        slot(i, i % 4, fire_next=False)
        i = n_chunk - 1
        ib = i % 4
        db = ib % 2
        wait_g(ib, db)
        fire_scat(ib, db)
        wait_scat((ib - 1) % 4, 1 - db)
        wait_scat(ib, db)

        plsc.subcore_barrier()

        for j in range(nz):
            pltpu.sync_copy(acc.at[pl.ds(row0 + j * KZ, KZ)], zchunk)
            pltpu.sync_copy(zchunk, out_hbm.at[c, pl.ds(row0 + j * KZ, KZ)])

    return scat


def _degree_kernel(N_pad, E):
    e_w = E // NW
    n_chunk = e_w // K
    words_per_tile = N_pad // NS

    mesh = plsc.VectorSubcoreMesh(core_axis_name="c", subcore_axis_name="s")

    @functools.partial(
        pl.kernel,
        out_type=jax.ShapeDtypeStruct((NC, N_pad), jnp.float32),
        mesh=mesh,
        scratch_types=[
            pltpu.VMEM_SHARED((N_pad,), jnp.float32),
            pltpu.VMEM((K,), jnp.int32),
            pltpu.VMEM((K,), jnp.float32),
            pltpu.VMEM((words_per_tile,), jnp.float32),
        ],
    )
    def degk(dst_hbm, ones_hbm, zeros_hbm, out_hbm, acc, dstv, onesv, zbuf):
        c = lax.axis_index("c")
        s = lax.axis_index("s")
        w0 = s * words_per_tile

        pltpu.sync_copy(zeros_hbm, zbuf)
        pltpu.sync_copy(zbuf, acc.at[pl.ds(w0, words_per_tile)])
        pltpu.sync_copy(ones_hbm, onesv)
        plsc.subcore_barrier()

        base = (c * NS + s) * e_w

        def body(i, _):
            off = base + i * K
            pltpu.sync_copy(dst_hbm.at[pl.ds(off, K)], dstv)
            pltpu.sync_copy(onesv, acc.at[dstv], add=True)
            return _

        lax.fori_loop(0, n_chunk, body, None)
        plsc.subcore_barrier()

        pltpu.sync_copy(acc.at[pl.ds(w0, words_per_tile)], zbuf)
        pltpu.sync_copy(zbuf, out_hbm.at[c, pl.ds(w0, words_per_tile)])

    return degk


def _pre_tc(N, D, N_pad):
    def body(cnt_ref, x_ref, w_ref, dinv_ref, u_ref):
        deg = cnt_ref[0, :N] + cnt_ref[1, :N] + 1.0
        dinv = lax.rsqrt(deg)
        dinv_ref[...] = dinv
        xw = jnp.dot(x_ref[...], w_ref[...], preferred_element_type=jnp.float32)
        u_ref[...] = xw * dinv[:, None]

    return pl.pallas_call(
        body,
        out_shape=(
            jax.ShapeDtypeStruct((N,), jnp.float32),
            jax.ShapeDtypeStruct((N, D), jnp.float32),
        ),
    )


def _layer_tc(N, D, N_pad, last):
    eps = 1e-5

    def body(S_ref, u_ref, dinv_ref, b_ref, g_ref, be_ref, w_ref, out_ref):
        u = u_ref[...]
        dinv = dinv_ref[...]
        S = S_ref[0, :N, :] + S_ref[1, :N, :] + u
        agg = S * dinv[:, None] + b_ref[...]
        mean = jnp.mean(agg, axis=0)
        var = jnp.mean((agg - mean[None, :]) ** 2, axis=0)
        h = (agg - mean[None, :]) * lax.rsqrt(var + eps)
        h = h * g_ref[...] + be_ref[...]
        h = jnp.maximum(h, 0.0)
        if last:
            out_ref[...] = h
        else:
            hw = jnp.dot(h, w_ref[...], preferred_element_type=jnp.float32)
            out_ref[...] = hw * dinv[:, None]

    return pl.pallas_call(
        body,
        out_shape=jax.ShapeDtypeStruct((N, D), jnp.float32),
    )


def kernel(x, edge_index, W, b, gamma, beta):
    N, D = x.shape
    E = edge_index.shape[1]
    L = W.shape[0]
    assert E % (NW * K) == 0
    N_pad = ((N + NS * K - 1) // (NS * K)) * (NS * K)

    src = edge_index[0]
    dst = edge_index[1]
    zeros_rows = jnp.zeros((KZ, D), jnp.float32)
    zeros_deg = jnp.zeros((N_pad // NS,), jnp.float32)
    ones_k = jnp.ones((K,), jnp.float32)

    n_chunk = (E + NW * K2 - 1) // (NW * K2)
    while n_chunk % 4 != 3:
        n_chunk += 1
    n_arr = ((n_chunk + 7) // 8) * 8
    E_proc = NW * K2 * n_chunk
    pad_iota = jnp.arange(E_proc - E, dtype=jnp.int32)
    src_pad = jnp.concatenate([src, pad_iota % N]).reshape(NW, n_chunk, K2)
    dst_pad = jnp.concatenate(
        [dst, N + pad_iota % (N_pad - N)]).reshape(NW, n_chunk, K2)
    if n_arr > n_chunk:
        extra = jnp.arange(NW * (n_arr - n_chunk) * K2, dtype=jnp.int32)
        extra = extra.reshape(NW, n_arr - n_chunk, K2)
        src_pad = jnp.concatenate([src_pad, extra % N], axis=1)
        dst_pad = jnp.concatenate([dst_pad, N + extra % (N_pad - N)], axis=1)

    cnt = _degree_kernel(N_pad, E)(dst, ones_k, zeros_deg)
    dinv, u = _pre_tc(N, D, N_pad)(cnt, x, W[0])

    scat = _scatter_rows_kernel(N_pad, D, (n_chunk, n_arr))
    for i in range(L):
        S = scat(u, src_pad, dst_pad, zeros_rows)
        layer = _layer_tc(N, D, N_pad, last=(i == L - 1))
        w_next = W[i + 1] if i < L - 1 else W[0]
        u = layer(S, u, dinv, b[i].reshape(1, D), gamma[i].reshape(1, D),
                  beta[i].reshape(1, D), w_next)
    return u


# speedup vs baseline: 26.3235x; 1.2281x over previous
import functools

import jax
import jax.numpy as jnp
from jax import lax
from jax.experimental import pallas as pl
from jax.experimental.pallas import tpu as pltpu
from jax.experimental.pallas import tpu_sc as plsc

NC = 2
NS = 16
NW = NC * NS
K = 80
K2 = 128


def _scatter_rows_kernel(N_pad, D, E_pad):
    n_chunk, n_arr = E_pad
    stripe = ((N_pad + NS * 8 - 1) // (NS * 8)) * 8
    zch = [K2] * (stripe // K2) + ([stripe % K2] if stripe % K2 else [])
    assert all(z % 8 == 0 for z in zch) and N_pad % 8 == 0
    assert n_chunk >= 8 and n_arr >= n_chunk
    UNROLL = 12
    n_peel = (n_chunk - 4) % UNROLL

    mesh = plsc.VectorSubcoreMesh(core_axis_name="c", subcore_axis_name="s")

    @functools.partial(
        pl.kernel,
        out_type=jax.ShapeDtypeStruct((NC, N_pad, D), jnp.float32),
        mesh=mesh,
        scratch_types=[
            pltpu.VMEM_SHARED((N_pad, D), jnp.float32),
            pltpu.VMEM((4, K2), jnp.int32),
            pltpu.VMEM((4, K2), jnp.int32),
            pltpu.VMEM((3, K2, D), jnp.float32),
            pltpu.SemaphoreType.DMA,
            pltpu.SemaphoreType.DMA,
            pltpu.SemaphoreType.DMA,
            pltpu.SemaphoreType.DMA,
            pltpu.SemaphoreType.DMA,
            pltpu.SemaphoreType.DMA,
            pltpu.SemaphoreType.DMA,
            pltpu.SemaphoreType.DMA,
            pltpu.SemaphoreType.DMA,
        ],
    )
    def scat(u_hbm, src_hbm, dst_hbm, zeros_hbm, out_hbm,
             acc, sidx, didx, ring,
             gs0, gs1, gs2, ss0, ss1, is0, is1, is2, is3):
        gsem = (gs0, gs1, gs2)
        ssem = (ss0, ss1)
        isem = (is0, is1, is2, is3)
        c = lax.axis_index("c")
        s = lax.axis_index("s")
        wid = c * NS + s
        row0 = pl.multiple_of(jnp.minimum(s * stripe, N_pad - stripe), 8)

        pltpu.sync_copy(zeros_hbm, ring.at[0])
        off = 0
        for z in zch:
            pltpu.sync_copy(ring.at[0, pl.ds(0, z)],
                            acc.at[pl.ds(row0 + off, z)])
            off += z
        plsc.subcore_barrier()

        def fire_idx(i, ib):
            pltpu.async_copy(src_hbm.at[wid, i], sidx.at[ib], isem[ib])
            pltpu.async_copy(dst_hbm.at[wid, i], didx.at[ib], isem[ib])

        def wait_idx(i, ib):
            pltpu.make_async_copy(src_hbm.at[wid, i], sidx.at[ib],
                                  isem[ib]).wait()
            pltpu.make_async_copy(dst_hbm.at[wid, i], didx.at[ib],
                                  isem[ib]).wait()

        def fire_g(ib, rb):
            pltpu.async_copy(u_hbm.at[sidx.at[ib]], ring.at[rb], gsem[rb])

        def wait_g(ib, rb):
            pltpu.make_async_copy(u_hbm.at[sidx.at[ib]], ring.at[rb],
                                  gsem[rb]).wait()

        def fire_scat(ib, rb, sb):
            pltpu.async_copy(ring.at[rb], acc.at[didx.at[ib]], ssem[sb],
                             add=True)

        def wait_scat(ib, rb, sb):
            pltpu.make_async_copy(ring.at[rb], acc.at[didx.at[ib]],
                                  ssem[sb]).wait()

        def slot(i, k, wait_prev=True, g_ahead=True, i_ahead=True):
            ib, rb, sb = k % 4, k % 3, k % 2
            wait_g(ib, rb)
            fire_scat(ib, rb, sb)
            if wait_prev:
                wait_scat((k - 1) % 4, (k - 1) % 3, (k - 1) % 2)
            if g_ahead:
                wait_idx(i + 2, (k + 2) % 4)
                fire_g((k + 2) % 4, (k + 2) % 3)
            if i_ahead:
                fire_idx(i + 3, (k + 3) % 4)

        fire_idx(0, 0)
        fire_idx(1, 1)
        fire_idx(2, 2)
        wait_idx(0, 0)
        fire_g(0, 0)
        wait_idx(1, 1)
        fire_g(1, 1)
        slot(0, 0, wait_prev=False)
        for i in range(1, 1 + n_peel):
            slot(i, i)

        base = 1 + n_peel

        def outer(g, _):
            for bb in range(UNROLL):
                slot(base + UNROLL * g + bb, base + bb)
            return _

        lax.fori_loop(0, (n_chunk - 4 - n_peel) // UNROLL, outer, None)
        slot(n_chunk - 3, n_chunk - 3, i_ahead=False)
        slot(n_chunk - 2, n_chunk - 2, g_ahead=False, i_ahead=False)
        i = n_chunk - 1
        slot(i, i, g_ahead=False, i_ahead=False)
        wait_scat(i % 4, i % 3, i % 2)

        plsc.subcore_barrier()

        off = 0
        for z in zch:
            pltpu.sync_copy(acc.at[pl.ds(row0 + off, z)],
                            ring.at[0, pl.ds(0, z)])
            pltpu.sync_copy(ring.at[0, pl.ds(0, z)],
                            out_hbm.at[c, pl.ds(row0 + off, z)])
            off += z

    return scat


def _degree_kernel(N_pad, E):
    e_w = E // NW
    n_chunk = e_w // K
    words_per_tile = N_pad // NS

    mesh = plsc.VectorSubcoreMesh(core_axis_name="c", subcore_axis_name="s")

    @functools.partial(
        pl.kernel,
        out_type=jax.ShapeDtypeStruct((NC, N_pad), jnp.float32),
        mesh=mesh,
        scratch_types=[
            pltpu.VMEM_SHARED((N_pad,), jnp.float32),
            pltpu.VMEM((K,), jnp.int32),
            pltpu.VMEM((K,), jnp.float32),
            pltpu.VMEM((words_per_tile,), jnp.float32),
        ],
    )
    def degk(dst_hbm, ones_hbm, zeros_hbm, out_hbm, acc, dstv, onesv, zbuf):
        c = lax.axis_index("c")
        s = lax.axis_index("s")
        w0 = s * words_per_tile

        pltpu.sync_copy(zeros_hbm, zbuf)
        pltpu.sync_copy(zbuf, acc.at[pl.ds(w0, words_per_tile)])
        pltpu.sync_copy(ones_hbm, onesv)
        plsc.subcore_barrier()

        base = (c * NS + s) * e_w

        def body(i, _):
            off = base + i * K
            pltpu.sync_copy(dst_hbm.at[pl.ds(off, K)], dstv)
            pltpu.sync_copy(onesv, acc.at[dstv], add=True)
            return _

        lax.fori_loop(0, n_chunk, body, None)
        plsc.subcore_barrier()

        pltpu.sync_copy(acc.at[pl.ds(w0, words_per_tile)], zbuf)
        pltpu.sync_copy(zbuf, out_hbm.at[c, pl.ds(w0, words_per_tile)])

    return degk


def _pre_tc(N, D, N_pad):
    def body(cnt_ref, x_ref, w_ref, dinv_ref, u_ref):
        deg = cnt_ref[0, :N] + cnt_ref[1, :N] + 1.0
        dinv = lax.rsqrt(deg)
        dinv_ref[...] = dinv
        xw = jnp.dot(x_ref[...], w_ref[...], preferred_element_type=jnp.float32)
        u_ref[...] = xw * dinv[:, None]

    return pl.pallas_call(
        body,
        out_shape=(
            jax.ShapeDtypeStruct((N,), jnp.float32),
            jax.ShapeDtypeStruct((N, D), jnp.float32),
        ),
    )


def _layer_tc(N, D, N_pad, last):
    eps = 1e-5

    def body(S_ref, u_ref, dinv_ref, b_ref, g_ref, be_ref, w_ref, out_ref):
        u = u_ref[...]
        dinv = dinv_ref[...]
        S = S_ref[0, :N, :] + S_ref[1, :N, :] + u
        agg = S * dinv[:, None] + b_ref[...]
        mean = jnp.mean(agg, axis=0)
        var = jnp.mean((agg - mean[None, :]) ** 2, axis=0)
        h = (agg - mean[None, :]) * lax.rsqrt(var + eps)
        h = h * g_ref[...] + be_ref[...]
        h = jnp.maximum(h, 0.0)
        if last:
            out_ref[...] = h
        else:
            hw = jnp.dot(h, w_ref[...], preferred_element_type=jnp.float32)
            out_ref[...] = hw * dinv[:, None]

    return pl.pallas_call(
        body,
        out_shape=jax.ShapeDtypeStruct((N, D), jnp.float32),
    )


def kernel(x, edge_index, W, b, gamma, beta):
    N, D = x.shape
    E = edge_index.shape[1]
    L = W.shape[0]
    assert E % (NW * K) == 0
    N_pad_deg = ((N + NS * K - 1) // (NS * K)) * (NS * K)
    N_pad = ((N + 7) // 8) * 8 + 104

    src = edge_index[0]
    dst = edge_index[1]
    zeros_rows = jnp.zeros((K2, D), jnp.float32)
    zeros_deg = jnp.zeros((N_pad_deg // NS,), jnp.float32)
    ones_k = jnp.ones((K,), jnp.float32)

    n_chunk = (E + NW * K2 - 1) // (NW * K2)
    n_arr = ((n_chunk + 7) // 8) * 8
    E_proc = NW * K2 * n_chunk
    pad_iota = jnp.arange(E_proc - E, dtype=jnp.int32)
    src_pad = jnp.concatenate([src, pad_iota % N]).reshape(NW, n_chunk, K2)
    dst_pad = jnp.concatenate(
        [dst, N + pad_iota % (N_pad - N)]).reshape(NW, n_chunk, K2)
    if n_arr > n_chunk:
        extra = jnp.arange(NW * (n_arr - n_chunk) * K2, dtype=jnp.int32)
        extra = extra.reshape(NW, n_arr - n_chunk, K2)
        src_pad = jnp.concatenate([src_pad, extra % N], axis=1)
        dst_pad = jnp.concatenate([dst_pad, N + extra % (N_pad - N)], axis=1)

    cnt = _degree_kernel(N_pad_deg, E)(dst, ones_k, zeros_deg)
    dinv, u = _pre_tc(N, D, N_pad_deg)(cnt, x, W[0])

    scat = _scatter_rows_kernel(N_pad, D, (n_chunk, n_arr))
    for i in range(L):
        S = scat(u, src_pad, dst_pad, zeros_rows)
        layer = _layer_tc(N, D, N_pad, last=(i == L - 1))
        w_next = W[i + 1] if i < L - 1 else W[0]
        u = layer(S, u, dinv, b[i].reshape(1, D), gamma[i].reshape(1, D),
                  beta[i].reshape(1, D), w_next)
    return u


# speedup vs baseline: 28.7387x; 1.0918x over previous
import functools

import jax
import jax.numpy as jnp
from jax import lax
from jax.experimental import pallas as pl
from jax.experimental.pallas import tpu as pltpu
from jax.experimental.pallas import tpu_sc as plsc

NC = 2
NS = 16
NW = NC * NS
K2 = 128


def _scatter_rows_kernel(N_pad, D, E_pad):
    n_chunk, n_arr = E_pad
    stripe = ((N_pad + NS * 8 - 1) // (NS * 8)) * 8
    zch = [K2] * (stripe // K2) + ([stripe % K2] if stripe % K2 else [])
    assert all(z % 8 == 0 for z in zch) and N_pad % 8 == 0
    assert n_chunk >= 8 and n_arr >= n_chunk
    UNROLL = 12
    n_peel = (n_chunk - 4) % UNROLL

    mesh = plsc.VectorSubcoreMesh(core_axis_name="c", subcore_axis_name="s")

    @functools.partial(
        pl.kernel,
        out_type=jax.ShapeDtypeStruct((NC, N_pad, D), jnp.float32),
        mesh=mesh,
        scratch_types=[
            pltpu.VMEM_SHARED((N_pad, D), jnp.float32),
            pltpu.VMEM((4, K2), jnp.int32),
            pltpu.VMEM((4, K2), jnp.int32),
            pltpu.VMEM((3, K2, D), jnp.float32),
            pltpu.SemaphoreType.DMA,
            pltpu.SemaphoreType.DMA,
            pltpu.SemaphoreType.DMA,
            pltpu.SemaphoreType.DMA,
            pltpu.SemaphoreType.DMA,
            pltpu.SemaphoreType.DMA,
            pltpu.SemaphoreType.DMA,
            pltpu.SemaphoreType.DMA,
            pltpu.SemaphoreType.DMA,
        ],
    )
    def scat(u_hbm, src_hbm, dst_hbm, zeros_hbm, out_hbm,
             acc, sidx, didx, ring,
             gs0, gs1, gs2, ss0, ss1, is0, is1, is2, is3):
        gsem = (gs0, gs1, gs2)
        ssem = (ss0, ss1)
        isem = (is0, is1, is2, is3)
        c = lax.axis_index("c")
        s = lax.axis_index("s")
        wid = c * NS + s
        row0 = pl.multiple_of(jnp.minimum(s * stripe, N_pad - stripe), 8)

        pltpu.sync_copy(zeros_hbm, ring.at[0])
        off = 0
        for z in zch:
            pltpu.sync_copy(ring.at[0, pl.ds(0, z)],
                            acc.at[pl.ds(row0 + off, z)])
            off += z
        plsc.subcore_barrier()

        def fire_idx(i, ib):
            pltpu.async_copy(src_hbm.at[wid, i], sidx.at[ib], isem[ib])
            pltpu.async_copy(dst_hbm.at[wid, i], didx.at[ib], isem[ib])

        def wait_idx(i, ib):
            pltpu.make_async_copy(src_hbm.at[wid, i], sidx.at[ib],
                                  isem[ib]).wait()
            pltpu.make_async_copy(dst_hbm.at[wid, i], didx.at[ib],
                                  isem[ib]).wait()

        def fire_g(ib, rb):
            pltpu.async_copy(u_hbm.at[sidx.at[ib]], ring.at[rb], gsem[rb])

        def wait_g(ib, rb):
            pltpu.make_async_copy(u_hbm.at[sidx.at[ib]], ring.at[rb],
                                  gsem[rb]).wait()

        def fire_scat(ib, rb, sb):
            pltpu.async_copy(ring.at[rb], acc.at[didx.at[ib]], ssem[sb],
                             add=True)

        def wait_scat(ib, rb, sb):
            pltpu.make_async_copy(ring.at[rb], acc.at[didx.at[ib]],
                                  ssem[sb]).wait()

        def slot(i, k, wait_prev=True, g_ahead=True, i_ahead=True):
            ib, rb, sb = k % 4, k % 3, k % 2
            wait_g(ib, rb)
            fire_scat(ib, rb, sb)
            if wait_prev:
                wait_scat((k - 1) % 4, (k - 1) % 3, (k - 1) % 2)
            if g_ahead:
                wait_idx(i + 2, (k + 2) % 4)
                fire_g((k + 2) % 4, (k + 2) % 3)
            if i_ahead:
                fire_idx(i + 3, (k + 3) % 4)

        fire_idx(0, 0)
        fire_idx(1, 1)
        fire_idx(2, 2)
        wait_idx(0, 0)
        fire_g(0, 0)
        wait_idx(1, 1)
        fire_g(1, 1)
        slot(0, 0, wait_prev=False)
        for i in range(1, 1 + n_peel):
            slot(i, i)

        base = 1 + n_peel

        def outer(g, _):
            for bb in range(UNROLL):
                slot(base + UNROLL * g + bb, base + bb)
            return _

        lax.fori_loop(0, (n_chunk - 4 - n_peel) // UNROLL, outer, None)
        slot(n_chunk - 3, n_chunk - 3, i_ahead=False)
        slot(n_chunk - 2, n_chunk - 2, g_ahead=False, i_ahead=False)
        i = n_chunk - 1
        slot(i, i, g_ahead=False, i_ahead=False)
        wait_scat(i % 4, i % 3, i % 2)

        plsc.subcore_barrier()

        off = 0
        for z in zch:
            pltpu.sync_copy(acc.at[pl.ds(row0 + off, z)],
                            ring.at[0, pl.ds(0, z)])
            pltpu.sync_copy(ring.at[0, pl.ds(0, z)],
                            out_hbm.at[c, pl.ds(row0 + off, z)])
            off += z

    return scat


def _degree_kernel(N_pad, n_chunks):
    n_chunk, n_arr = n_chunks
    words_per_tile = N_pad // NS
    assert words_per_tile % 8 == 0 and n_chunk >= 8
    UNROLL = 4
    n_peel = (n_chunk - 4) % UNROLL

    mesh = plsc.VectorSubcoreMesh(core_axis_name="c", subcore_axis_name="s")

    @functools.partial(
        pl.kernel,
        out_type=jax.ShapeDtypeStruct((NC, N_pad), jnp.float32),
        mesh=mesh,
        scratch_types=[
            pltpu.VMEM_SHARED((N_pad,), jnp.float32),
            pltpu.VMEM((4, K2), jnp.int32),
            pltpu.VMEM((K2,), jnp.float32),
            pltpu.VMEM((words_per_tile,), jnp.float32),
            pltpu.SemaphoreType.DMA,
            pltpu.SemaphoreType.DMA,
            pltpu.SemaphoreType.DMA,
            pltpu.SemaphoreType.DMA,
            pltpu.SemaphoreType.DMA,
            pltpu.SemaphoreType.DMA,
        ],
    )
    def degk(dst_hbm, ones_hbm, zeros_hbm, out_hbm, acc, didx, onesv, zbuf,
             ss0, ss1, is0, is1, is2, is3):
        ssem = (ss0, ss1)
        isem = (is0, is1, is2, is3)
        c = lax.axis_index("c")
        s = lax.axis_index("s")
        wid = c * NS + s
        w0 = s * words_per_tile

        pltpu.sync_copy(zeros_hbm, zbuf)
        pltpu.sync_copy(zbuf, acc.at[pl.ds(w0, words_per_tile)])
        pltpu.sync_copy(ones_hbm, onesv)
        plsc.subcore_barrier()

        def fire_idx(i, ib):
            pltpu.async_copy(dst_hbm.at[wid, i], didx.at[ib], isem[ib])

        def wait_idx(i, ib):
            pltpu.make_async_copy(dst_hbm.at[wid, i], didx.at[ib],
                                  isem[ib]).wait()

        def fire_scat(ib, sb):
            pltpu.async_copy(onesv, acc.at[didx.at[ib]], ssem[sb], add=True)

        def wait_scat(ib, sb):
            pltpu.make_async_copy(onesv, acc.at[didx.at[ib]],
                                  ssem[sb]).wait()

        def slot(i, k, wait_prev=True, i_ahead=True):
            ib, sb = k % 4, k % 2
            wait_idx(i, ib)
            fire_scat(ib, sb)
            if wait_prev:
                wait_scat((k - 1) % 4, (k - 1) % 2)
            if i_ahead:
                fire_idx(i + 3, (k + 3) % 4)

        fire_idx(0, 0)
        fire_idx(1, 1)
        fire_idx(2, 2)
        slot(0, 0, wait_prev=False)
        for i in range(1, 1 + n_peel):
            slot(i, i)

        base = 1 + n_peel

        def outer(g, _):
            for bb in range(UNROLL):
                slot(base + UNROLL * g + bb, base + bb)
            return _

        lax.fori_loop(0, (n_chunk - 4 - n_peel) // UNROLL, outer, None)
        for i in range(n_chunk - 3, n_chunk):
            slot(i, i, i_ahead=False)
        wait_scat((n_chunk - 1) % 4, (n_chunk - 1) % 2)

        plsc.subcore_barrier()

        pltpu.sync_copy(acc.at[pl.ds(w0, words_per_tile)], zbuf)
        pltpu.sync_copy(zbuf, out_hbm.at[c, pl.ds(w0, words_per_tile)])

    return degk


def _pre_tc(N, D, N_pad):
    def body(cnt_ref, x_ref, w_ref, dinv_ref, u_ref):
        deg = cnt_ref[0, :N] + cnt_ref[1, :N] + 1.0
        dinv = lax.rsqrt(deg)
        dinv_ref[...] = dinv
        xw = jnp.dot(x_ref[...], w_ref[...], preferred_element_type=jnp.float32)
        u_ref[...] = xw * dinv[:, None]

    return pl.pallas_call(
        body,
        out_shape=(
            jax.ShapeDtypeStruct((N,), jnp.float32),
            jax.ShapeDtypeStruct((N, D), jnp.float32),
        ),
    )


def _layer_tc(N, D, N_pad, last):
    eps = 1e-5

    def body(S_ref, u_ref, dinv_ref, b_ref, g_ref, be_ref, w_ref, out_ref):
        u = u_ref[...]
        dinv = dinv_ref[...]
        S = S_ref[0, :N, :] + S_ref[1, :N, :] + u
        agg = S * dinv[:, None] + b_ref[...]
        mean = jnp.mean(agg, axis=0)
        var = jnp.mean((agg - mean[None, :]) ** 2, axis=0)
        h = (agg - mean[None, :]) * lax.rsqrt(var + eps)
        h = h * g_ref[...] + be_ref[...]
        h = jnp.maximum(h, 0.0)
        if last:
            out_ref[...] = h
        else:
            hw = jnp.dot(h, w_ref[...], preferred_element_type=jnp.float32)
            out_ref[...] = hw * dinv[:, None]

    return pl.pallas_call(
        body,
        out_shape=jax.ShapeDtypeStruct((N, D), jnp.float32),
    )


def kernel(x, edge_index, W, b, gamma, beta):
    N, D = x.shape
    E = edge_index.shape[1]
    L = W.shape[0]
    N_pad_deg = ((N + NS * 128 - 1) // (NS * 128)) * (NS * 128)
    N_pad = ((N + 7) // 8) * 8 + 104

    src = edge_index[0]
    dst = edge_index[1]
    zeros_rows = jnp.zeros((K2, D), jnp.float32)
    zeros_deg = jnp.zeros((N_pad_deg // NS,), jnp.float32)
    ones_k = jnp.ones((K2,), jnp.float32)

    n_chunk = (E + NW * K2 - 1) // (NW * K2)
    n_arr = ((n_chunk + 7) // 8) * 8
    E_proc = NW * K2 * n_chunk
    pad_iota = jnp.arange(E_proc - E, dtype=jnp.int32)
    src_pad = jnp.concatenate([src, pad_iota % N]).reshape(NW, n_chunk, K2)
    dst_pad = jnp.concatenate(
        [dst, N + pad_iota % (N_pad - N)]).reshape(NW, n_chunk, K2)
    if n_arr > n_chunk:
        extra = jnp.arange(NW * (n_arr - n_chunk) * K2, dtype=jnp.int32)
        extra = extra.reshape(NW, n_arr - n_chunk, K2)
        src_pad = jnp.concatenate([src_pad, extra % N], axis=1)
        dst_pad = jnp.concatenate([dst_pad, N + extra % (N_pad - N)], axis=1)

    cnt = _degree_kernel(N_pad_deg, (n_chunk, n_arr))(dst_pad, ones_k, zeros_deg)
    dinv, u = _pre_tc(N, D, N_pad_deg)(cnt, x, W[0])

    scat = _scatter_rows_kernel(N_pad, D, (n_chunk, n_arr))
    for i in range(L):
        S = scat(u, src_pad, dst_pad, zeros_rows)
        layer = _layer_tc(N, D, N_pad, last=(i == L - 1))
        w_next = W[i + 1] if i < L - 1 else W[0]
        u = layer(S, u, dinv, b[i].reshape(1, D), gamma[i].reshape(1, D),
                  beta[i].reshape(1, D), w_next)
    return u


# speedup vs baseline: 28.8516x; 1.0039x over previous
import functools

import jax
import jax.numpy as jnp
from jax import lax
from jax.experimental import pallas as pl
from jax.experimental.pallas import tpu as pltpu
from jax.experimental.pallas import tpu_sc as plsc

NC = 2
NS = 16
NW = NC * NS
K2 = 128


def _scatter_rows_kernel(N_pad, D, E_pad):
    n_chunk, n_arr = E_pad
    stripe = ((N_pad + NS * 8 - 1) // (NS * 8)) * 8
    assert N_pad % 8 == 0
    assert n_chunk >= 8 and n_arr >= n_chunk
    UNROLL = 12
    n_peel = (n_chunk - 4) % UNROLL

    mesh = plsc.VectorSubcoreMesh(core_axis_name="c", subcore_axis_name="s")

    @functools.partial(
        pl.kernel,
        out_type=jax.ShapeDtypeStruct((NC, N_pad, D), jnp.float32),
        mesh=mesh,
        scratch_types=[
            pltpu.VMEM_SHARED((N_pad, D), jnp.float32),
            pltpu.VMEM((4, K2), jnp.int32),
            pltpu.VMEM((4, K2), jnp.int32),
            pltpu.VMEM((3, K2, D), jnp.float32),
            pltpu.SemaphoreType.DMA,
            pltpu.SemaphoreType.DMA,
            pltpu.SemaphoreType.DMA,
            pltpu.SemaphoreType.DMA,
            pltpu.SemaphoreType.DMA,
            pltpu.SemaphoreType.DMA,
            pltpu.SemaphoreType.DMA,
            pltpu.SemaphoreType.DMA,
            pltpu.SemaphoreType.DMA,
        ],
    )
    def scat(u_hbm, src_hbm, dst_hbm, zeros_hbm, out_hbm,
             acc, sidx, didx, ring,
             gs0, gs1, gs2, ss0, ss1, is0, is1, is2, is3):
        gsem = (gs0, gs1, gs2)
        ssem = (ss0, ss1)
        isem = (is0, is1, is2, is3)
        c = lax.axis_index("c")
        s = lax.axis_index("s")
        wid = c * NS + s
        row0 = pl.multiple_of(jnp.minimum(s * stripe, N_pad - stripe), 8)

        pltpu.sync_copy(zeros_hbm, acc.at[pl.ds(row0, stripe)])
        plsc.subcore_barrier()

        def fire_idx(i, ib):
            pltpu.async_copy(src_hbm.at[wid, i], sidx.at[ib], isem[ib])
            pltpu.async_copy(dst_hbm.at[wid, i], didx.at[ib], isem[ib])

        def wait_idx(i, ib):
            pltpu.make_async_copy(src_hbm.at[wid, i], sidx.at[ib],
                                  isem[ib]).wait()
            pltpu.make_async_copy(dst_hbm.at[wid, i], didx.at[ib],
                                  isem[ib]).wait()

        def fire_g(ib, rb):
            pltpu.async_copy(u_hbm.at[sidx.at[ib]], ring.at[rb], gsem[rb])

        def wait_g(ib, rb):
            pltpu.make_async_copy(u_hbm.at[sidx.at[ib]], ring.at[rb],
                                  gsem[rb]).wait()

        def fire_scat(ib, rb, sb):
            pltpu.async_copy(ring.at[rb], acc.at[didx.at[ib]], ssem[sb],
                             add=True)

        def wait_scat(ib, rb, sb):
            pltpu.make_async_copy(ring.at[rb], acc.at[didx.at[ib]],
                                  ssem[sb]).wait()

        def slot(i, k, wait_prev=True, g_ahead=True, i_ahead=True):
            ib, rb, sb = k % 4, k % 3, k % 2
            wait_g(ib, rb)
            fire_scat(ib, rb, sb)
            if wait_prev:
                wait_scat((k - 1) % 4, (k - 1) % 3, (k - 1) % 2)
            if g_ahead:
                wait_idx(i + 2, (k + 2) % 4)
                fire_g((k + 2) % 4, (k + 2) % 3)
            if i_ahead:
                fire_idx(i + 3, (k + 3) % 4)

        fire_idx(0, 0)
        fire_idx(1, 1)
        fire_idx(2, 2)
        wait_idx(0, 0)
        fire_g(0, 0)
        wait_idx(1, 1)
        fire_g(1, 1)
        slot(0, 0, wait_prev=False)
        for i in range(1, 1 + n_peel):
            slot(i, i)

        base = 1 + n_peel

        def outer(g, _):
            for bb in range(UNROLL):
                slot(base + UNROLL * g + bb, base + bb)
            return _

        lax.fori_loop(0, (n_chunk - 4 - n_peel) // UNROLL, outer, None)
        slot(n_chunk - 3, n_chunk - 3, i_ahead=False)
        slot(n_chunk - 2, n_chunk - 2, g_ahead=False, i_ahead=False)
        i = n_chunk - 1
        slot(i, i, g_ahead=False, i_ahead=False)
        wait_scat(i % 4, i % 3, i % 2)

        plsc.subcore_barrier()

        pltpu.sync_copy(acc.at[pl.ds(row0, stripe)],
                        out_hbm.at[c, pl.ds(row0, stripe)])

    return scat


def _degree_kernel(N_pad, n_chunks):
    n_chunk, n_arr = n_chunks
    words_per_tile = N_pad // NS
    assert words_per_tile % 8 == 0 and n_chunk >= 8
    UNROLL = 4
    n_peel = (n_chunk - 4) % UNROLL

    mesh = plsc.VectorSubcoreMesh(core_axis_name="c", subcore_axis_name="s")

    @functools.partial(
        pl.kernel,
        out_type=jax.ShapeDtypeStruct((NC, N_pad), jnp.float32),
        mesh=mesh,
        scratch_types=[
            pltpu.VMEM_SHARED((N_pad,), jnp.float32),
            pltpu.VMEM((4, K2), jnp.int32),
            pltpu.VMEM((K2,), jnp.float32),
            pltpu.VMEM((words_per_tile,), jnp.float32),
            pltpu.SemaphoreType.DMA,
            pltpu.SemaphoreType.DMA,
            pltpu.SemaphoreType.DMA,
            pltpu.SemaphoreType.DMA,
            pltpu.SemaphoreType.DMA,
            pltpu.SemaphoreType.DMA,
        ],
    )
    def degk(dst_hbm, ones_hbm, zeros_hbm, out_hbm, acc, didx, onesv, zbuf,
             ss0, ss1, is0, is1, is2, is3):
        ssem = (ss0, ss1)
        isem = (is0, is1, is2, is3)
        c = lax.axis_index("c")
        s = lax.axis_index("s")
        wid = c * NS + s
        w0 = s * words_per_tile

        pltpu.sync_copy(zeros_hbm, zbuf)
        pltpu.sync_copy(zbuf, acc.at[pl.ds(w0, words_per_tile)])
        pltpu.sync_copy(ones_hbm, onesv)
        plsc.subcore_barrier()

        def fire_idx(i, ib):
            pltpu.async_copy(dst_hbm.at[wid, i], didx.at[ib], isem[ib])

        def wait_idx(i, ib):
            pltpu.make_async_copy(dst_hbm.at[wid, i], didx.at[ib],
                                  isem[ib]).wait()

        def fire_scat(ib, sb):
            pltpu.async_copy(onesv, acc.at[didx.at[ib]], ssem[sb], add=True)

        def wait_scat(ib, sb):
            pltpu.make_async_copy(onesv, acc.at[didx.at[ib]],
                                  ssem[sb]).wait()

        def slot(i, k, wait_prev=True, i_ahead=True):
            ib, sb = k % 4, k % 2
            wait_idx(i, ib)
            fire_scat(ib, sb)
            if wait_prev:
                wait_scat((k - 1) % 4, (k - 1) % 2)
            if i_ahead:
                fire_idx(i + 3, (k + 3) % 4)

        fire_idx(0, 0)
        fire_idx(1, 1)
        fire_idx(2, 2)
        slot(0, 0, wait_prev=False)
        for i in range(1, 1 + n_peel):
            slot(i, i)

        base = 1 + n_peel

        def outer(g, _):
            for bb in range(UNROLL):
                slot(base + UNROLL * g + bb, base + bb)
            return _

        lax.fori_loop(0, (n_chunk - 4 - n_peel) // UNROLL, outer, None)
        for i in range(n_chunk - 3, n_chunk):
            slot(i, i, i_ahead=False)
        wait_scat((n_chunk - 1) % 4, (n_chunk - 1) % 2)

        plsc.subcore_barrier()

        pltpu.sync_copy(acc.at[pl.ds(w0, words_per_tile)], zbuf)
        pltpu.sync_copy(zbuf, out_hbm.at[c, pl.ds(w0, words_per_tile)])

    return degk


def _pre_tc(N, D, N_pad):
    def body(cnt_ref, x_ref, w_ref, dinv_ref, u_ref):
        deg = cnt_ref[0, :N] + cnt_ref[1, :N] + 1.0
        dinv = lax.rsqrt(deg)
        dinv_ref[...] = dinv
        xw = jnp.dot(x_ref[...], w_ref[...], preferred_element_type=jnp.float32)
        u_ref[...] = xw * dinv[:, None]

    return pl.pallas_call(
        body,
        out_shape=(
            jax.ShapeDtypeStruct((N,), jnp.float32),
            jax.ShapeDtypeStruct((N, D), jnp.float32),
        ),
    )


def _layer_tc(N, D, N_pad, last):
    eps = 1e-5

    def body(S_ref, u_ref, dinv_ref, b_ref, g_ref, be_ref, w_ref, out_ref):
        u = u_ref[...]
        dinv = dinv_ref[...]
        S = S_ref[0, :N, :] + S_ref[1, :N, :] + u
        agg = S * dinv[:, None] + b_ref[...]
        mean = jnp.mean(agg, axis=0)
        var = jnp.mean((agg - mean[None, :]) ** 2, axis=0)
        h = (agg - mean[None, :]) * lax.rsqrt(var + eps)
        h = h * g_ref[...] + be_ref[...]
        h = jnp.maximum(h, 0.0)
        if last:
            out_ref[...] = h
        else:
            hw = jnp.dot(h, w_ref[...], preferred_element_type=jnp.float32)
            out_ref[...] = hw * dinv[:, None]

    return pl.pallas_call(
        body,
        out_shape=jax.ShapeDtypeStruct((N, D), jnp.float32),
    )


def kernel(x, edge_index, W, b, gamma, beta):
    N, D = x.shape
    E = edge_index.shape[1]
    L = W.shape[0]
    N_pad_deg = ((N + NS * 128 - 1) // (NS * 128)) * (NS * 128)
    N_pad = ((N + 7) // 8) * 8 + 104

    src = edge_index[0]
    dst = edge_index[1]
    stripe = ((N_pad + NS * 8 - 1) // (NS * 8)) * 8
    zeros_rows = jnp.zeros((stripe, D), jnp.float32)
    zeros_deg = jnp.zeros((N_pad_deg // NS,), jnp.float32)
    ones_k = jnp.ones((K2,), jnp.float32)

    n_chunk = (E + NW * K2 - 1) // (NW * K2)
    n_arr = ((n_chunk + 7) // 8) * 8
    E_proc = NW * K2 * n_chunk
    pad_iota = jnp.arange(E_proc - E, dtype=jnp.int32)
    src_pad = jnp.concatenate([src, pad_iota % N]).reshape(NW, n_chunk, K2)
    dst_pad = jnp.concatenate(
        [dst, N + pad_iota % (N_pad - N)]).reshape(NW, n_chunk, K2)
    if n_arr > n_chunk:
        extra = jnp.arange(NW * (n_arr - n_chunk) * K2, dtype=jnp.int32)
        extra = extra.reshape(NW, n_arr - n_chunk, K2)
        src_pad = jnp.concatenate([src_pad, extra % N], axis=1)
        dst_pad = jnp.concatenate([dst_pad, N + extra % (N_pad - N)], axis=1)

    cnt = _degree_kernel(N_pad_deg, (n_chunk, n_arr))(dst_pad, ones_k, zeros_deg)
    dinv, u = _pre_tc(N, D, N_pad_deg)(cnt, x, W[0])

    scat = _scatter_rows_kernel(N_pad, D, (n_chunk, n_arr))
    for i in range(L):
        S = scat(u, src_pad, dst_pad, zeros_rows)
        layer = _layer_tc(N, D, N_pad, last=(i == L - 1))
        w_next = W[i + 1] if i < L - 1 else W[0]
        u = layer(S, u, dinv, b[i].reshape(1, D), gamma[i].reshape(1, D),
                  beta[i].reshape(1, D), w_next)
    return u


# speedup vs baseline: 29.3247x; 1.0164x over previous
import functools

import jax
import jax.numpy as jnp
from jax import lax
from jax.experimental import pallas as pl
from jax.experimental.pallas import tpu as pltpu
from jax.experimental.pallas import tpu_sc as plsc

NC = 2
NS = 16
NW = NC * NS
K2 = 128


def _scatter_rows_kernel(N_pad, D, E_pad):
    n_chunk, n_arr = E_pad
    stripe = ((N_pad + NS * 8 - 1) // (NS * 8)) * 8
    assert N_pad % 8 == 0
    assert n_chunk >= 8 and n_arr >= n_chunk
    UNROLL = 12
    n_peel = (n_chunk - 4) % UNROLL

    mesh = plsc.VectorSubcoreMesh(core_axis_name="c", subcore_axis_name="s")

    @functools.partial(
        pl.kernel,
        out_type=jax.ShapeDtypeStruct((NC, N_pad, D), jnp.float32),
        mesh=mesh,
        scratch_types=[
            pltpu.VMEM_SHARED((N_pad, D), jnp.float32),
            pltpu.VMEM((4, K2), jnp.int32),
            pltpu.VMEM((4, K2), jnp.int32),
            pltpu.VMEM((3, K2, D), jnp.float32),
            pltpu.SemaphoreType.DMA,
            pltpu.SemaphoreType.DMA,
            pltpu.SemaphoreType.DMA,
            pltpu.SemaphoreType.DMA,
            pltpu.SemaphoreType.DMA,
            pltpu.SemaphoreType.DMA,
            pltpu.SemaphoreType.DMA,
            pltpu.SemaphoreType.DMA,
            pltpu.SemaphoreType.DMA,
            pltpu.SemaphoreType.DMA,
        ],
    )
    def scat(u_hbm, src_hbm, dst_hbm, zeros_hbm, out_hbm,
             acc, sidx, didx, ring,
             gs0, gs1, gs2, ss0, ss1, is0, is1, is2, is3, zsem):
        gsem = (gs0, gs1, gs2)
        ssem = (ss0, ss1)
        isem = (is0, is1, is2, is3)
        c = lax.axis_index("c")
        s = lax.axis_index("s")
        wid = c * NS + s
        row0 = pl.multiple_of(jnp.minimum(s * stripe, N_pad - stripe), 8)

        zcopy = pltpu.make_async_copy(zeros_hbm, acc.at[pl.ds(row0, stripe)],
                                      zsem)
        zcopy.start()

        def fire_idx(i, ib):
            pltpu.async_copy(src_hbm.at[wid, i], sidx.at[ib], isem[ib])
            pltpu.async_copy(dst_hbm.at[wid, i], didx.at[ib], isem[ib])

        def wait_idx(i, ib):
            pltpu.make_async_copy(src_hbm.at[wid, i], sidx.at[ib],
                                  isem[ib]).wait()
            pltpu.make_async_copy(dst_hbm.at[wid, i], didx.at[ib],
                                  isem[ib]).wait()

        def fire_g(ib, rb):
            pltpu.async_copy(u_hbm.at[sidx.at[ib]], ring.at[rb], gsem[rb])

        def wait_g(ib, rb):
            pltpu.make_async_copy(u_hbm.at[sidx.at[ib]], ring.at[rb],
                                  gsem[rb]).wait()

        def fire_scat(ib, rb, sb):
            pltpu.async_copy(ring.at[rb], acc.at[didx.at[ib]], ssem[sb],
                             add=True)

        def wait_scat(ib, rb, sb):
            pltpu.make_async_copy(ring.at[rb], acc.at[didx.at[ib]],
                                  ssem[sb]).wait()

        def slot(i, k, wait_prev=True, g_ahead=True, i_ahead=True):
            ib, rb, sb = k % 4, k % 3, k % 2
            wait_g(ib, rb)
            fire_scat(ib, rb, sb)
            if wait_prev:
                wait_scat((k - 1) % 4, (k - 1) % 3, (k - 1) % 2)
            if g_ahead:
                wait_idx(i + 2, (k + 2) % 4)
                fire_g((k + 2) % 4, (k + 2) % 3)
            if i_ahead:
                fire_idx(i + 3, (k + 3) % 4)

        fire_idx(0, 0)
        fire_idx(1, 1)
        fire_idx(2, 2)
        wait_idx(0, 0)
        fire_g(0, 0)
        wait_idx(1, 1)
        fire_g(1, 1)
        zcopy.wait()
        plsc.subcore_barrier()
        slot(0, 0, wait_prev=False)
        for i in range(1, 1 + n_peel):
            slot(i, i)

        base = 1 + n_peel

        def outer(g, _):
            for bb in range(UNROLL):
                slot(base + UNROLL * g + bb, base + bb)
            return _

        lax.fori_loop(0, (n_chunk - 4 - n_peel) // UNROLL, outer, None)
        slot(n_chunk - 3, n_chunk - 3, i_ahead=False)
        slot(n_chunk - 2, n_chunk - 2, g_ahead=False, i_ahead=False)
        i = n_chunk - 1
        slot(i, i, g_ahead=False, i_ahead=False)
        wait_scat(i % 4, i % 3, i % 2)

        plsc.subcore_barrier()

        pltpu.sync_copy(acc.at[pl.ds(row0, stripe)],
                        out_hbm.at[c, pl.ds(row0, stripe)])

    return scat


def _degree_kernel(N_pad, n_chunks):
    n_chunk, n_arr = n_chunks
    words_per_tile = N_pad // NS
    assert words_per_tile % 8 == 0 and n_chunk >= 8
    UNROLL = 4
    n_peel = (n_chunk - 4) % UNROLL

    mesh = plsc.VectorSubcoreMesh(core_axis_name="c", subcore_axis_name="s")

    @functools.partial(
        pl.kernel,
        out_type=jax.ShapeDtypeStruct((NC, N_pad), jnp.float32),
        mesh=mesh,
        scratch_types=[
            pltpu.VMEM_SHARED((N_pad,), jnp.float32),
            pltpu.VMEM((4, K2), jnp.int32),
            pltpu.VMEM((K2,), jnp.float32),
            pltpu.VMEM((words_per_tile,), jnp.float32),
            pltpu.SemaphoreType.DMA,
            pltpu.SemaphoreType.DMA,
            pltpu.SemaphoreType.DMA,
            pltpu.SemaphoreType.DMA,
            pltpu.SemaphoreType.DMA,
            pltpu.SemaphoreType.DMA,
        ],
    )
    def degk(dst_hbm, ones_hbm, zeros_hbm, out_hbm, acc, didx, onesv, zbuf,
             ss0, ss1, is0, is1, is2, is3):
        ssem = (ss0, ss1)
        isem = (is0, is1, is2, is3)
        c = lax.axis_index("c")
        s = lax.axis_index("s")
        wid = c * NS + s
        w0 = s * words_per_tile

        pltpu.sync_copy(zeros_hbm, zbuf)
        pltpu.sync_copy(zbuf, acc.at[pl.ds(w0, words_per_tile)])
        pltpu.sync_copy(ones_hbm, onesv)
        plsc.subcore_barrier()

        def fire_idx(i, ib):
            pltpu.async_copy(dst_hbm.at[wid, i], didx.at[ib], isem[ib])

        def wait_idx(i, ib):
            pltpu.make_async_copy(dst_hbm.at[wid, i], didx.at[ib],
                                  isem[ib]).wait()

        def fire_scat(ib, sb):
            pltpu.async_copy(onesv, acc.at[didx.at[ib]], ssem[sb], add=True)

        def wait_scat(ib, sb):
            pltpu.make_async_copy(onesv, acc.at[didx.at[ib]],
                                  ssem[sb]).wait()

        def slot(i, k, wait_prev=True, i_ahead=True):
            ib, sb = k % 4, k % 2
            wait_idx(i, ib)
            fire_scat(ib, sb)
            if wait_prev:
                wait_scat((k - 1) % 4, (k - 1) % 2)
            if i_ahead:
                fire_idx(i + 3, (k + 3) % 4)

        fire_idx(0, 0)
        fire_idx(1, 1)
        fire_idx(2, 2)
        slot(0, 0, wait_prev=False)
        for i in range(1, 1 + n_peel):
            slot(i, i)

        base = 1 + n_peel

        def outer(g, _):
            for bb in range(UNROLL):
                slot(base + UNROLL * g + bb, base + bb)
            return _

        lax.fori_loop(0, (n_chunk - 4 - n_peel) // UNROLL, outer, None)
        for i in range(n_chunk - 3, n_chunk):
            slot(i, i, i_ahead=False)
        wait_scat((n_chunk - 1) % 4, (n_chunk - 1) % 2)

        plsc.subcore_barrier()

        pltpu.sync_copy(acc.at[pl.ds(w0, words_per_tile)], zbuf)
        pltpu.sync_copy(zbuf, out_hbm.at[c, pl.ds(w0, words_per_tile)])

    return degk


def _pre_tc(N, D, N_pad):
    def body(cnt_ref, x_ref, w_ref, dinv_ref, u_ref):
        deg = cnt_ref[0, :N] + cnt_ref[1, :N] + 1.0
        dinv = lax.rsqrt(deg)
        dinv_ref[...] = dinv
        xw = jnp.dot(x_ref[...], w_ref[...], preferred_element_type=jnp.float32)
        u_ref[...] = xw * dinv[:, None]

    return pl.pallas_call(
        body,
        out_shape=(
            jax.ShapeDtypeStruct((N,), jnp.float32),
            jax.ShapeDtypeStruct((N, D), jnp.float32),
        ),
    )


def _layer_tc(N, D, N_pad, last):
    eps = 1e-5

    def body(S_ref, u_ref, dinv_ref, b_ref, g_ref, be_ref, w_ref, out_ref):
        u = u_ref[...]
        dinv = dinv_ref[...]
        S = S_ref[0, :N, :] + S_ref[1, :N, :] + u
        agg = S * dinv[:, None] + b_ref[...]
        mean = jnp.mean(agg, axis=0)
        var = jnp.mean((agg - mean[None, :]) ** 2, axis=0)
        h = (agg - mean[None, :]) * lax.rsqrt(var + eps)
        h = h * g_ref[...] + be_ref[...]
        h = jnp.maximum(h, 0.0)
        if last:
            out_ref[...] = h
        else:
            hw = jnp.dot(h, w_ref[...], preferred_element_type=jnp.float32)
            out_ref[...] = hw * dinv[:, None]

    return pl.pallas_call(
        body,
        out_shape=jax.ShapeDtypeStruct((N, D), jnp.float32),
    )


def kernel(x, edge_index, W, b, gamma, beta):
    N, D = x.shape
    E = edge_index.shape[1]
    L = W.shape[0]
    N_pad_deg = ((N + NS * 128 - 1) // (NS * 128)) * (NS * 128)
    N_pad = ((N + 7) // 8) * 8 + 104

    src = edge_index[0]
    dst = edge_index[1]
    stripe = ((N_pad + NS * 8 - 1) // (NS * 8)) * 8
    zeros_rows = jnp.zeros((stripe, D), jnp.float32)
    zeros_deg = jnp.zeros((N_pad_deg // NS,), jnp.float32)
    ones_k = jnp.ones((K2,), jnp.float32)

    n_chunk = (E + NW * K2 - 1) // (NW * K2)
    n_arr = ((n_chunk + 7) // 8) * 8
    E_proc = NW * K2 * n_chunk
    pad_iota = jnp.arange(E_proc - E, dtype=jnp.int32)
    src_pad = jnp.concatenate([src, pad_iota % N]).reshape(NW, n_chunk, K2)
    dst_pad = jnp.concatenate(
        [dst, N + pad_iota % (N_pad - N)]).reshape(NW, n_chunk, K2)
    if n_arr > n_chunk:
        extra = jnp.arange(NW * (n_arr - n_chunk) * K2, dtype=jnp.int32)
        extra = extra.reshape(NW, n_arr - n_chunk, K2)
        src_pad = jnp.concatenate([src_pad, extra % N], axis=1)
        dst_pad = jnp.concatenate([dst_pad, N + extra % (N_pad - N)], axis=1)

    cnt = _degree_kernel(N_pad_deg, (n_chunk, n_arr))(dst_pad, ones_k, zeros_deg)
    dinv, u = _pre_tc(N, D, N_pad_deg)(cnt, x, W[0])

    scat = _scatter_rows_kernel(N_pad, D, (n_chunk, n_arr))
    for i in range(L):
        S = scat(u, src_pad, dst_pad, zeros_rows)
        layer = _layer_tc(N, D, N_pad, last=(i == L - 1))
        w_next = W[i + 1] if i < L - 1 else W[0]
        u = layer(S, u, dinv, b[i].reshape(1, D), gamma[i].reshape(1, D),
                  beta[i].reshape(1, D), w_next)
    return u
